# initial kernel scaffold (unmeasured)
import jax
import jax.numpy as jnp
from jax import lax
from jax.experimental import pallas as pl
from jax.experimental.pallas import tpu as pltpu

N_DEV = 4
SQ = 2048
SKV = 2048
H_PER = 8
DH = 128
DM = 1024
QC = 4
QCH = SQ // QC
CH = SQ // N_DEV
SCALE = 0.08838834764831843
BLK = 64


def kernel(x, Wq, K_ext, V_ext, Wo):
    my = lax.axis_index("i")
    K_s = lax.dynamic_slice_in_dim(K_ext, my * H_PER, H_PER, axis=2)
    V_s = lax.dynamic_slice_in_dim(V_ext, my * H_PER, H_PER, axis=2)

    def body(x_ref, wq_ref, k_ref, v_ref, wo_ref, out_ref,
             bias_ref, rs_recv, send_sems, recv_sems):
        qc = pl.program_id(0)
        h = pl.program_id(1)
        my_pos = lax.axis_index("i")
        left = lax.rem(my_pos - 1 + N_DEV, N_DEV)
        right = lax.rem(my_pos + 1, N_DEV)

        @pl.when(jnp.logical_and(qc == 0, h == 0))
        def _entry_barrier():
            barrier = pltpu.get_barrier_semaphore()
            for nbr in (left, right):
                pl.semaphore_signal(
                    barrier, inc=1, device_id=(nbr,),
                    device_id_type=pl.DeviceIdType.MESH,
                )
            pl.semaphore_wait(barrier, 2)

        @pl.when(h == 0)
        def _make_bias():
            rowb = (lax.broadcasted_iota(jnp.int32, (QCH, SKV), 0) + qc * QCH) // BLK
            colb = lax.broadcasted_iota(jnp.int32, (QCH, SKV), 1) // BLK
            keep = (rowb == colb) | (colb == 0) | (lax.rem(rowb + colb, 3) == 0)
            bias_ref[...] = jnp.where(keep, 0.0, -1e9).astype(jnp.float32)

        xc = x_ref[0]
        kh = k_ref[0, :, 0, :]
        vh = v_ref[0, :, 0, :]
        qh = jnp.dot(xc, wq_ref[...], preferred_element_type=jnp.float32)
        sc = lax.dot_general(qh, kh, (((1,), (1,)), ((), ())),
                             preferred_element_type=jnp.float32)
        sc = sc * SCALE + bias_ref[...]
        m = jnp.max(sc, axis=1, keepdims=True)
        w = jnp.exp(sc - m)
        denom = jnp.sum(w, axis=1, keepdims=True)
        ctx = jnp.dot(w, vh, preferred_element_type=jnp.float32) / denom
        contrib = jnp.dot(ctx, wo_ref[...], preferred_element_type=jnp.float32)

        rows = pl.ds(qc * QCH, QCH)

        @pl.when(h == 0)
        def _init():
            out_ref[rows, :] = contrib

        @pl.when(h > 0)
        def _accum():
            out_ref[rows, :] += contrib

        @pl.when(jnp.logical_and(qc == QC - 1, h == H_PER - 1))
        def _collective():
            for st in range(N_DEV - 1):
                send_c = lax.rem(my_pos - st + N_DEV, N_DEV)
                recv_c = lax.rem(my_pos - st - 1 + N_DEV, N_DEV)
                rdma = pltpu.make_async_remote_copy(
                    src_ref=out_ref.at[pl.ds(send_c * CH, CH), :],
                    dst_ref=rs_recv.at[st],
                    send_sem=send_sems.at[st],
                    recv_sem=recv_sems.at[st],
                    device_id=(right,),
                    device_id_type=pl.DeviceIdType.MESH,
                )
                rdma.start()
                rdma.wait()
                out_ref[pl.ds(recv_c * CH, CH), :] += rs_recv[st]
            for t in range(N_DEV - 1):
                g = lax.rem(my_pos + 1 - t + N_DEV, N_DEV)
                rdma = pltpu.make_async_remote_copy(
                    src_ref=out_ref.at[pl.ds(g * CH, CH), :],
                    dst_ref=out_ref.at[pl.ds(g * CH, CH), :],
                    send_sem=send_sems.at[N_DEV - 1 + t],
                    recv_sem=recv_sems.at[N_DEV - 1 + t],
                    device_id=(right,),
                    device_id_type=pl.DeviceIdType.MESH,
                )
                rdma.start()
                rdma.wait()

    out = pl.pallas_call(
        body,
        grid=(QC, H_PER),
        in_specs=[
            pl.BlockSpec((1, QCH, DM), lambda qc, h: (0, qc, 0)),
            pl.BlockSpec((DM, DH), lambda qc, h: (0, h)),
            pl.BlockSpec((1, SKV, 1, DH), lambda qc, h: (0, 0, h, 0)),
            pl.BlockSpec((1, SKV, 1, DH), lambda qc, h: (0, 0, h, 0)),
            pl.BlockSpec((DH, DM), lambda qc, h: (h, 0)),
        ],
        out_specs=pl.BlockSpec((SQ, DM), lambda qc, h: (0, 0)),
        out_shape=jax.ShapeDtypeStruct((SQ, DM), jnp.float32),
        scratch_shapes=[
            pltpu.VMEM((QCH, SKV), jnp.float32),
            pltpu.VMEM((N_DEV - 1, CH, DM), jnp.float32),
            pltpu.SemaphoreType.DMA((2 * (N_DEV - 1),)),
            pltpu.SemaphoreType.DMA((2 * (N_DEV - 1),)),
        ],
        compiler_params=pltpu.CompilerParams(collective_id=0),
    )(x, Wq, K_s, V_s, Wo)
    return out.reshape(1, SQ, DM)


# baseline (device time: 310803 ns/iter reference)
import jax
import jax.numpy as jnp
from jax import lax
from jax.experimental import pallas as pl
from jax.experimental.pallas import tpu as pltpu

N_DEV = 4
SQ = 2048
SKV = 2048
H_PER = 8
DH = 128
DM = 1024
QC = 4
QCH = SQ // QC
CH = SQ // N_DEV
SCALE = 0.08838834764831843
BLK = 64


def kernel(x, Wq, K_ext, V_ext, Wo):
    my = lax.axis_index("i")
    K_s = lax.dynamic_slice_in_dim(K_ext, my * H_PER, H_PER, axis=2)[0].transpose(1, 0, 2)
    V_s = lax.dynamic_slice_in_dim(V_ext, my * H_PER, H_PER, axis=2)[0].transpose(1, 0, 2)

    def body(x_ref, wq_ref, k_ref, v_ref, wo_ref, out_ref,
             bias_ref, rs_recv, send_sems, recv_sems):
        qc = pl.program_id(0)
        h = pl.program_id(1)
        my_pos = lax.axis_index("i")
        left = lax.rem(my_pos - 1 + N_DEV, N_DEV)
        right = lax.rem(my_pos + 1, N_DEV)

        @pl.when(jnp.logical_and(qc == 0, h == 0))
        def _entry_barrier():
            barrier = pltpu.get_barrier_semaphore()
            for nbr in (left, right):
                pl.semaphore_signal(
                    barrier, inc=1, device_id=(nbr,),
                    device_id_type=pl.DeviceIdType.MESH,
                )
            pl.semaphore_wait(barrier, 2)

        @pl.when(h == 0)
        def _make_bias():
            rowb = (lax.broadcasted_iota(jnp.int32, (QCH, SKV), 0) + qc * QCH) // BLK
            colb = lax.broadcasted_iota(jnp.int32, (QCH, SKV), 1) // BLK
            keep = (rowb == colb) | (colb == 0) | (lax.rem(rowb + colb, 3) == 0)
            bias_ref[...] = jnp.where(keep, 0.0, -1e9).astype(jnp.float32)

        xc = x_ref[0]
        kh = k_ref[0]
        vh = v_ref[0]
        qh = jnp.dot(xc, wq_ref[...], preferred_element_type=jnp.float32)
        sc = lax.dot_general(qh, kh, (((1,), (1,)), ((), ())),
                             preferred_element_type=jnp.float32)
        sc = sc * SCALE + bias_ref[...]
        m = jnp.max(sc, axis=1, keepdims=True)
        w = jnp.exp(sc - m)
        denom = jnp.sum(w, axis=1, keepdims=True)
        ctx = jnp.dot(w, vh, preferred_element_type=jnp.float32) / denom
        contrib = jnp.dot(ctx, wo_ref[...], preferred_element_type=jnp.float32)

        rows = pl.ds(qc * QCH, QCH)

        @pl.when(h == 0)
        def _init():
            out_ref[rows, :] = contrib

        @pl.when(h > 0)
        def _accum():
            out_ref[rows, :] += contrib

        @pl.when(jnp.logical_and(qc == QC - 1, h == H_PER - 1))
        def _collective():
            for st in range(N_DEV - 1):
                send_c = lax.rem(my_pos - st + N_DEV, N_DEV)
                recv_c = lax.rem(my_pos - st - 1 + N_DEV, N_DEV)
                rdma = pltpu.make_async_remote_copy(
                    src_ref=out_ref.at[pl.ds(send_c * CH, CH), :],
                    dst_ref=rs_recv.at[st],
                    send_sem=send_sems.at[st],
                    recv_sem=recv_sems.at[st],
                    device_id=(right,),
                    device_id_type=pl.DeviceIdType.MESH,
                )
                rdma.start()
                rdma.wait()
                out_ref[pl.ds(recv_c * CH, CH), :] += rs_recv[st]
            for t in range(N_DEV - 1):
                g = lax.rem(my_pos + 1 - t + N_DEV, N_DEV)
                rdma = pltpu.make_async_remote_copy(
                    src_ref=out_ref.at[pl.ds(g * CH, CH), :],
                    dst_ref=out_ref.at[pl.ds(g * CH, CH), :],
                    send_sem=send_sems.at[N_DEV - 1 + t],
                    recv_sem=recv_sems.at[N_DEV - 1 + t],
                    device_id=(right,),
                    device_id_type=pl.DeviceIdType.MESH,
                )
                rdma.start()
                rdma.wait()

    out = pl.pallas_call(
        body,
        grid=(QC, H_PER),
        in_specs=[
            pl.BlockSpec((1, QCH, DM), lambda qc, h: (0, qc, 0)),
            pl.BlockSpec((DM, DH), lambda qc, h: (0, h)),
            pl.BlockSpec((1, SKV, DH), lambda qc, h: (h, 0, 0)),
            pl.BlockSpec((1, SKV, DH), lambda qc, h: (h, 0, 0)),
            pl.BlockSpec((DH, DM), lambda qc, h: (h, 0)),
        ],
        out_specs=pl.BlockSpec((SQ, DM), lambda qc, h: (0, 0)),
        out_shape=jax.ShapeDtypeStruct((SQ, DM), jnp.float32),
        scratch_shapes=[
            pltpu.VMEM((QCH, SKV), jnp.float32),
            pltpu.VMEM((N_DEV - 1, CH, DM), jnp.float32),
            pltpu.SemaphoreType.DMA((2 * (N_DEV - 1),)),
            pltpu.SemaphoreType.DMA((2 * (N_DEV - 1),)),
        ],
        compiler_params=pltpu.CompilerParams(collective_id=0),
    )(x, Wq, K_s, V_s, Wo)
    return out.reshape(1, SQ, DM)


# device time: 207324 ns/iter; 1.4991x vs baseline; 1.4991x over previous
import jax
import jax.numpy as jnp
from jax import lax
from jax.experimental import pallas as pl
from jax.experimental.pallas import tpu as pltpu

N_DEV = 4
SQ = 2048
SKV = 2048
H_PER = 8
DH = 128
DM = 1024
QC = 4
CH = SQ // N_DEV
HCH = CH // 2
SCALE = 0.08838834764831843
BLK = 64
MESH = pl.DeviceIdType.MESH


def kernel(x, Wq, K_ext, V_ext, Wo):
    my = lax.axis_index("i")
    K_s = lax.dynamic_slice_in_dim(K_ext, my * H_PER, H_PER, axis=2)[0].transpose(1, 0, 2)
    V_s = lax.dynamic_slice_in_dim(V_ext, my * H_PER, H_PER, axis=2)[0].transpose(1, 0, 2)
    xb = x.astype(jnp.bfloat16)
    Wqb = Wq.astype(jnp.bfloat16)
    Kb = K_s.astype(jnp.bfloat16)
    Vb = V_s.astype(jnp.bfloat16)
    Wob = Wo.astype(jnp.bfloat16)

    def body(x_ref, wq_ref, k_ref, v_ref, wo_ref, out_ref,
             bias_ref, rs_recv, send_sems, recv_sems):
        qc = pl.program_id(0)
        h = pl.program_id(1)
        my_pos = lax.axis_index("i")
        left = lax.rem(my_pos - 1 + N_DEV, N_DEV)
        right = lax.rem(my_pos + 1, N_DEV)
        chunk = lax.rem(my_pos - qc + N_DEV, N_DEV)
        rows = pl.ds(chunk * CH, CH)

        @pl.when(jnp.logical_and(qc == 0, h == 0))
        def _entry_barrier():
            barrier = pltpu.get_barrier_semaphore()
            for nbr in (left, right):
                pl.semaphore_signal(barrier, inc=1, device_id=(nbr,),
                                    device_id_type=MESH)
            pl.semaphore_wait(barrier, 2)

        @pl.when(h == 0)
        def _make_bias():
            rowb = (lax.broadcasted_iota(jnp.int32, (CH, SKV), 0) + chunk * CH) // BLK
            colb = lax.broadcasted_iota(jnp.int32, (CH, SKV), 1) // BLK
            keep = (rowb == colb) | (colb == 0) | (lax.rem(rowb + colb, 3) == 0)
            bias_ref[...] = jnp.where(keep, 0.0, -1e9).astype(jnp.float32)

        xc = x_ref[0, rows, :]
        kh = k_ref[0]
        vh = v_ref[0]
        qh = jnp.dot(xc, wq_ref[...], preferred_element_type=jnp.float32)
        sc = lax.dot_general(qh.astype(jnp.bfloat16), kh,
                             (((1,), (1,)), ((), ())),
                             preferred_element_type=jnp.float32)
        sc = sc * SCALE + bias_ref[...]
        m = jnp.max(sc, axis=1, keepdims=True)
        w = jnp.exp(sc - m)
        denom = jnp.sum(w, axis=1, keepdims=True)
        ctx = jnp.dot(w.astype(jnp.bfloat16), vh,
                      preferred_element_type=jnp.float32) / denom
        contrib = jnp.dot(ctx.astype(jnp.bfloat16), wo_ref[...],
                          preferred_element_type=jnp.float32)

        @pl.when(h == 0)
        def _init():
            out_ref[rows, :] = contrib

        @pl.when(h > 0)
        def _accum():
            out_ref[rows, :] += contrib

        last_h = h == H_PER - 1

        @pl.when(jnp.logical_and(last_h, qc > 0))
        def _rs_recv_add():
            st = qc - 1
            rdma = pltpu.make_async_remote_copy(
                src_ref=out_ref.at[rows, :],
                dst_ref=rs_recv.at[st],
                send_sem=send_sems.at[st],
                recv_sem=recv_sems.at[st],
                device_id=(left,),
                device_id_type=MESH,
            )
            rdma.wait_recv()
            out_ref[rows, :] += rs_recv[st]

        @pl.when(jnp.logical_and(last_h, qc < QC - 1))
        def _rs_send():
            rdma = pltpu.make_async_remote_copy(
                src_ref=out_ref.at[rows, :],
                dst_ref=rs_recv.at[qc],
                send_sem=send_sems.at[qc],
                recv_sem=recv_sems.at[qc],
                device_id=(right,),
                device_id_type=MESH,
            )
            rdma.start()

        @pl.when(jnp.logical_and(last_h, qc == QC - 1))
        def _finish():
            for st in range(N_DEV - 1):
                src_c = lax.rem(my_pos - st + N_DEV, N_DEV)
                pltpu.make_async_remote_copy(
                    src_ref=out_ref.at[pl.ds(src_c * CH, CH), :],
                    dst_ref=rs_recv.at[st],
                    send_sem=send_sems.at[st],
                    recv_sem=recv_sems.at[st],
                    device_id=(right,),
                    device_id_type=MESH,
                ).wait_send()
            for t in range(N_DEV - 1):
                cw_c = lax.rem(my_pos + 1 - t + N_DEV, N_DEV)
                ccw_c = lax.rem(my_pos + 1 + t, N_DEV)
                cw = pltpu.make_async_remote_copy(
                    src_ref=out_ref.at[pl.ds(cw_c * CH, HCH), :],
                    dst_ref=out_ref.at[pl.ds(cw_c * CH, HCH), :],
                    send_sem=send_sems.at[3 + t],
                    recv_sem=recv_sems.at[3 + t],
                    device_id=(right,),
                    device_id_type=MESH,
                )
                ccw = pltpu.make_async_remote_copy(
                    src_ref=out_ref.at[pl.ds(ccw_c * CH + HCH, HCH), :],
                    dst_ref=out_ref.at[pl.ds(ccw_c * CH + HCH, HCH), :],
                    send_sem=send_sems.at[6 + t],
                    recv_sem=recv_sems.at[6 + t],
                    device_id=(left,),
                    device_id_type=MESH,
                )
                cw.start()
                ccw.start()
                cw.wait()
                ccw.wait()

    out = pl.pallas_call(
        body,
        grid=(QC, H_PER),
        in_specs=[
            pl.BlockSpec((1, SQ, DM), lambda qc, h: (0, 0, 0)),
            pl.BlockSpec((DM, DH), lambda qc, h: (0, h)),
            pl.BlockSpec((1, SKV, DH), lambda qc, h: (h, 0, 0)),
            pl.BlockSpec((1, SKV, DH), lambda qc, h: (h, 0, 0)),
            pl.BlockSpec((DH, DM), lambda qc, h: (h, 0)),
        ],
        out_specs=pl.BlockSpec((SQ, DM), lambda qc, h: (0, 0)),
        out_shape=jax.ShapeDtypeStruct((SQ, DM), jnp.float32),
        scratch_shapes=[
            pltpu.VMEM((CH, SKV), jnp.float32),
            pltpu.VMEM((N_DEV - 1, CH, DM), jnp.float32),
            pltpu.SemaphoreType.DMA((9,)),
            pltpu.SemaphoreType.DMA((9,)),
        ],
        compiler_params=pltpu.CompilerParams(collective_id=0),
    )(xb, Wqb, Kb, Vb, Wob)
    return out.reshape(1, SQ, DM)
